# baseline (speedup 1.0000x reference)
import functools

import jax
import jax.numpy as jnp
from jax import lax
from jax.experimental import pallas as pl
from jax.experimental.pallas import tpu as pltpu
from jax.experimental.pallas import tpu_sc as plsc

_N = 16384 * 3 * 2
_NW = 16
_CHUNK = _N // _NW
_NVREG = _CHUNK // 16
_PERIOD = 6


def _sc_body(x_hbm, w_hbm, out_hbm, xv, wv):
    wid = lax.axis_index("s")
    base = wid * _CHUNK
    pltpu.sync_copy(w_hbm, wv)
    pltpu.sync_copy(x_hbm.at[pl.ds(base, _CHUNK)], xv)
    bias = [wv[pl.ds(v * 16, 16)] for v in range(3)]

    def body(g, _):
        for v in range(3):
            sl = pl.ds(g * 48 + v * 16, 16)
            xv[sl] = xv[sl] + bias[v]
        return _

    lax.fori_loop(0, _CHUNK // 48, body, 0)
    pltpu.sync_copy(xv, out_hbm.at[pl.ds(base, _CHUNK)])


_sc_add = functools.partial(
    pl.kernel,
    out_type=jax.ShapeDtypeStruct((_N,), jnp.float32),
    mesh=plsc.VectorSubcoreMesh(core_axis_name="c", subcore_axis_name="s",
                                num_cores=1),
    scratch_types=[
        pltpu.VMEM((_CHUNK,), jnp.float32),
        pltpu.VMEM((48,), jnp.float32),
    ],
)(_sc_body)


def kernel(x, W768):
    wflat = W768[: x.shape[1]].reshape(-1)
    wtiled = jnp.tile(wflat, 48 // wflat.shape[0])
    out = _sc_add(x.reshape(-1), wtiled)
    return out.reshape(x.shape)


# speedup vs baseline: 1.0109x; 1.0109x over previous
import functools

import jax
import jax.numpy as jnp
from jax import lax
from jax.experimental import pallas as pl
from jax.experimental.pallas import tpu as pltpu
from jax.experimental.pallas import tpu_sc as plsc

_N = 16384 * 3 * 2
_NW = 16
_CHUNK = _N // _NW
_HALF = _CHUNK // 2
_PERIOD = 6


def _sc_body(x_hbm, w_hbm, out_hbm, xv, wv, isem, osem):
    base = lax.axis_index("s") * _CHUNK
    hw = pltpu.async_copy(w_hbm, wv, isem)
    h0 = pltpu.async_copy(x_hbm.at[pl.ds(base, _HALF)], xv.at[pl.ds(0, _HALF)],
                          isem)
    h1 = pltpu.async_copy(x_hbm.at[pl.ds(base + _HALF, _HALF)],
                          xv.at[pl.ds(_HALF, _HALF)], isem)
    hw.wait()
    bias = [wv[pl.ds(v * 16, 16)] for v in range(3)]

    def add_half(off):
        def body(g, carry):
            b0 = off + g * 96
            for v in range(6):
                sl = pl.ds(b0 + v * 16, 16)
                xv[sl] = xv[sl] + bias[v % 3]
            return carry

        lax.fori_loop(0, _HALF // 96, body, 0)

    h0.wait()
    add_half(0)
    o0 = pltpu.async_copy(xv.at[pl.ds(0, _HALF)],
                          out_hbm.at[pl.ds(base, _HALF)], osem)
    h1.wait()
    add_half(_HALF)
    o1 = pltpu.async_copy(xv.at[pl.ds(_HALF, _HALF)],
                          out_hbm.at[pl.ds(base + _HALF, _HALF)], osem)
    o0.wait()
    o1.wait()


_sc_add = functools.partial(
    pl.kernel,
    out_type=jax.ShapeDtypeStruct((_N,), jnp.float32),
    mesh=plsc.VectorSubcoreMesh(core_axis_name="c", subcore_axis_name="s",
                                num_cores=1),
    scratch_types=[
        pltpu.VMEM((_CHUNK,), jnp.float32),
        pltpu.VMEM((48,), jnp.float32),
        pltpu.SemaphoreType.DMA,
        pltpu.SemaphoreType.DMA,
    ],
)(_sc_body)


def kernel(x, W768):
    wflat = W768[: x.shape[1]].reshape(-1)
    wtiled = jnp.tile(wflat, 48 // wflat.shape[0])
    out = _sc_add(x.reshape(-1), wtiled)
    return out.reshape(x.shape)
